# Initial kernel scaffold; baseline (speedup 1.0000x reference)
import jax
import jax.numpy as jnp
from jax.experimental import pallas as pl


def kernel(node_locations, time_deadline, W0, b0):
    raise NotImplementedError("write your pallas kernel here")


# speedup vs baseline: 1.2938x; 1.2938x over previous
import jax
import jax.numpy as jnp
from jax.experimental import pallas as pl
from jax.experimental.pallas import tpu as pltpu

N_REAL = 2049
NP = 2304
THRESH2 = 0.04 * 0.04
BI = 768
NI = NP // BI
D = 128


def _adj_body(xc_ref, yc_ref, xr_ref, yr_ref, a_ref):
    xi = xc_ref[:, 0:1]
    yi = yc_ref[:, 0:1]
    xj = xr_ref[0:1, :]
    yj = yr_ref[0:1, :]
    dx = xi - xj
    dy = yi - yj
    d2 = dx * dx + dy * dy
    a_ref[...] = (d2 <= THRESH2).astype(jnp.bfloat16)


def _fv1_body(a_ref, xc_ref, yc_ref, tc_ref, w_ref, b_ref, out_ref):
    xk = xc_ref[:, 0:1]
    yk = yc_ref[:, 0:1]
    tk = tc_ref[:, 0:1]
    wx = w_ref[0:1, :]
    wy = w_ref[1:2, :]
    wt = w_ref[2:3, :]
    bb = b_ref[0:1, :]
    fv0 = jnp.maximum(xk * wx + yk * wy + tk * wt + bb, 0.0)
    hi = fv0.astype(jnp.bfloat16)
    lo = (fv0 - hi.astype(jnp.float32)).astype(jnp.bfloat16)
    a = a_ref[...]
    out_ref[...] = (
        jax.lax.dot(a, hi, preferred_element_type=jnp.float32)
        + jax.lax.dot(a, lo, preferred_element_type=jnp.float32))


def _m_body(a1_ref, a2_ref, m_ref):
    cnt = jax.lax.dot(a1_ref[...], a2_ref[...],
                      preferred_element_type=jnp.float32)
    m_ref[...] = (cnt > 0.5).astype(jnp.bfloat16)


def _fv2_body(m1_ref, a2_ref, mij_ref, fv1_ref, out_ref):
    j = pl.program_id(1)
    ma = jax.lax.dot(m1_ref[...], a2_ref[...],
                     preferred_element_type=jnp.float32)
    c = ma * mij_ref[...].astype(jnp.float32)
    contrib = jax.lax.dot(c, fv1_ref[...],
                          preferred_element_type=jnp.float32)

    @pl.when(j == 0)
    def _():
        out_ref[...] = contrib

    @pl.when(j > 0)
    def _():
        out_ref[...] += contrib


_adj = pl.pallas_call(
    _adj_body,
    grid=(NI, NI),
    in_specs=[
        pl.BlockSpec((BI, 128), lambda i, j: (i, 0)),
        pl.BlockSpec((BI, 128), lambda i, j: (i, 0)),
        pl.BlockSpec((8, BI), lambda i, j: (0, j)),
        pl.BlockSpec((8, BI), lambda i, j: (0, j)),
    ],
    out_specs=pl.BlockSpec((BI, BI), lambda i, j: (i, j)),
    out_shape=jax.ShapeDtypeStruct((NP, NP), jnp.bfloat16),
)

_fv1 = pl.pallas_call(
    _fv1_body,
    grid=(NI,),
    in_specs=[
        pl.BlockSpec((BI, NP), lambda i: (i, 0)),
        pl.BlockSpec((NP, 128), lambda i: (0, 0)),
        pl.BlockSpec((NP, 128), lambda i: (0, 0)),
        pl.BlockSpec((NP, 128), lambda i: (0, 0)),
        pl.BlockSpec((8, 128), lambda i: (0, 0)),
        pl.BlockSpec((8, 128), lambda i: (0, 0)),
    ],
    out_specs=pl.BlockSpec((BI, D), lambda i: (i, 0)),
    out_shape=jax.ShapeDtypeStruct((NP, D), jnp.float32),
)

_m = pl.pallas_call(
    _m_body,
    grid=(NI, NI),
    in_specs=[
        pl.BlockSpec((BI, NP), lambda i, j: (i, 0)),
        pl.BlockSpec((NP, BI), lambda i, j: (0, j)),
    ],
    out_specs=pl.BlockSpec((BI, BI), lambda i, j: (i, j)),
    out_shape=jax.ShapeDtypeStruct((NP, NP), jnp.bfloat16),
)

_fv2 = pl.pallas_call(
    _fv2_body,
    grid=(NI, NI),
    in_specs=[
        pl.BlockSpec((BI, NP), lambda i, j: (i, 0)),
        pl.BlockSpec((NP, BI), lambda i, j: (0, j)),
        pl.BlockSpec((BI, BI), lambda i, j: (i, j)),
        pl.BlockSpec((BI, D), lambda i, j: (j, 0)),
    ],
    out_specs=pl.BlockSpec((BI, D), lambda i, j: (i, 0)),
    out_shape=jax.ShapeDtypeStruct((NP, D), jnp.float32),
)


def kernel(node_locations, time_deadline, W0, b0):
    depot = jax.random.uniform(jax.random.key(1), (1, 2), dtype=jnp.float32)
    loc = jnp.concatenate([depot, node_locations], axis=0)
    tdc = jnp.concatenate(
        [jnp.zeros((1,), jnp.float32), time_deadline[:, 0]], axis=0)
    pad = NP - N_REAL
    x = jnp.concatenate([loc[:, 0], jnp.full((pad,), 1000.0, jnp.float32)])
    y = jnp.concatenate([loc[:, 1], jnp.full((pad,), 2000.0, jnp.float32)])
    t = jnp.concatenate([tdc, jnp.zeros((pad,), jnp.float32)])

    xc = jnp.broadcast_to(x[:, None], (NP, 128))
    yc = jnp.broadcast_to(y[:, None], (NP, 128))
    tc = jnp.broadcast_to(t[:, None], (NP, 128))
    xr = jnp.broadcast_to(x[None, :], (8, NP))
    yr = jnp.broadcast_to(y[None, :], (8, NP))

    wpad = jnp.zeros((8, 128), jnp.float32).at[0:3, :].set(W0.T)
    bpad = jnp.zeros((8, 128), jnp.float32).at[0, :].set(b0)

    a = _adj(xc, yc, xr, yr)
    fv1 = _fv1(a, xc, yc, tc, wpad, bpad)
    m = _m(a, a)
    fv2 = _fv2(m, a, m, fv1)
    return fv2[:N_REAL]
